# baseline (speedup 1.0000x reference)
import functools

import jax
import jax.numpy as jnp
from jax import lax
from jax.experimental import pallas as pl
from jax.experimental.pallas import tpu as pltpu
from jax.experimental.pallas import tpu_sc as plsc

N_VOCAB = 1000000
N_EMB = 64
NC = 2
NS = 16
NW = NC * NS
CHUNK = 128
NBUF = 5
NTB = 2


def _row_mapping(g):
    h = g // 4
    base = 2 * (lax.iota(jnp.int32, 16) + 16 * g - 64 * h)
    return h, base


def _gather_body(idx_hbm, emb_hbm, out_hbm, idx_v, rows_v, tb0, tb1, gsem, wsem):
    n_s = idx_hbm.shape[1]
    wid = lax.axis_index("s") * NC + lax.axis_index("c")
    pltpu.sync_copy(idx_hbm.at[wid], idx_v)

    def start_gathers(s, b):
        for h in range(2):
            pltpu.async_copy(
                emb_hbm.at[idx_v.at[s, h]], rows_v.at[b, h], gsem.at[b]
            )

    def wait_gathers(s, b):
        for h in range(2):
            pltpu.make_async_copy(
                emb_hbm.at[idx_v.at[s, h]], rows_v.at[b, h], gsem.at[b]
            ).wait()

    def out_slice(s):
        return out_hbm.at[s, :, wid]

    for b in range(NBUF):
        start_gathers(b, b)

    bases = [_row_mapping(g) for g in range(8)]

    @pl.loop(0, n_s, step=NBUF)
    def _(j):
        for b in range(NBUF):
            s = j + b
            tb = tb0 if b % 2 == 0 else tb1
            wait_gathers(s, b)

            @pl.when(s >= NTB)
            def _():
                pltpu.make_async_copy(tb, out_slice(s), wsem.at[b % 2]).wait()

            @pl.loop(0, 8)
            def _(tr):
                for sub in range(8):
                    e = 8 * tr + sub
                    d = e // 32
                    c = jnp.full((16,), e % 32, dtype=jnp.int32)
                    for g in range(8):
                        h, base = bases[g]
                        vec = plsc.load_gather(
                            rows_v.at[b, h], [base + d, c]
                        )
                        tb[tr, sub, pl.ds(16 * g, 16)] = vec

            pltpu.async_copy(tb, out_slice(s), wsem.at[b % 2])

            @pl.when(s + NBUF < n_s)
            def _():
                start_gathers(s + NBUF, b)

    for b2 in range(NTB):
        tb = tb0 if b2 == 0 else tb1
        pltpu.make_async_copy(tb, out_slice(0), wsem.at[b2]).wait()


def kernel(xs, emb):
    b, s = xs.shape
    assert b % CHUNK == 0 and (b // CHUNK) == NW
    i = xs.reshape(NW, CHUNK, s).transpose(0, 2, 1)
    idx2 = jnp.stack([2 * i, 2 * i + 1], axis=-1).reshape(NW, s, 2, CHUNK)
    emb2 = emb.reshape(2 * N_VOCAB, N_EMB // 2)

    mesh = plsc.VectorSubcoreMesh(
        core_axis_name="c", subcore_axis_name="s", num_cores=NC
    )
    run = functools.partial(
        pl.kernel,
        out_type=jax.ShapeDtypeStruct((s, 8, NW, 8, CHUNK), jnp.float32),
        mesh=mesh,
        scratch_types=[
            pltpu.VMEM((s, 2, CHUNK), jnp.int32),
            pltpu.VMEM((NBUF, 2, CHUNK, N_EMB // 2), jnp.float32),
            pltpu.VMEM((8, 8, CHUNK), jnp.float32),
            pltpu.VMEM((8, 8, CHUNK), jnp.float32),
            pltpu.SemaphoreType.DMA((NBUF,)),
            pltpu.SemaphoreType.DMA((NTB,)),
        ],
        compiler_params=pltpu.CompilerParams(
            use_tc_tiling_on_sc=False, needs_layout_passes=False
        ),
    )(_gather_body)
    out = run(idx2, emb2)
    return out.transpose(2, 4, 0, 1, 3).reshape(b, s, N_EMB)


# speedup vs baseline: 1.5081x; 1.5081x over previous
import functools

import jax
import jax.numpy as jnp
from jax import lax
from jax.experimental import pallas as pl
from jax.experimental.pallas import tpu as pltpu
from jax.experimental.pallas import tpu_sc as plsc

N_VOCAB = 1000000
N_EMB = 64
NC = 2
NS = 16
NW = NC * NS
CHUNK = 128


NBUF = 8


def _gather_body(idx_hbm, emb_hbm, out_hbm, idx_v, rows_v, gsem):
    k = idx_hbm.shape[1]
    wid = lax.axis_index("s") * NC + lax.axis_index("c")
    pltpu.sync_copy(idx_hbm.at[wid], idx_v)

    for b in range(NBUF):
        pltpu.async_copy(emb_hbm.at[idx_v.at[b]], rows_v.at[b], gsem.at[b])

    @pl.loop(0, k, step=NBUF)
    def _(j):
        for b in range(NBUF):
            g = j + b
            pltpu.make_async_copy(
                emb_hbm.at[idx_v.at[b]], rows_v.at[b], gsem.at[b]
            ).wait()
            pltpu.sync_copy(rows_v.at[b], out_hbm.at[wid, g])

            @pl.when(g + NBUF < k)
            def _():
                pltpu.async_copy(
                    emb_hbm.at[idx_v.at[g + NBUF]], rows_v.at[b], gsem.at[b]
                )


def kernel(xs, emb):
    b, s = xs.shape
    n = b * s
    assert n % (NW * CHUNK) == 0
    k = n // (NW * CHUNK)
    idx = xs.reshape(NW, k, CHUNK)

    mesh = plsc.VectorSubcoreMesh(
        core_axis_name="c", subcore_axis_name="s", num_cores=NC
    )
    run = functools.partial(
        pl.kernel,
        out_type=jax.ShapeDtypeStruct((NW, k, CHUNK, N_EMB), jnp.float32),
        mesh=mesh,
        scratch_types=[
            pltpu.VMEM((k, CHUNK), jnp.int32),
            pltpu.VMEM((NBUF, CHUNK, N_EMB), jnp.float32),
            pltpu.SemaphoreType.DMA((NBUF,)),
        ],
        compiler_params=pltpu.CompilerParams(
            use_tc_tiling_on_sc=False, skip_device_barrier=True
        ),
    )(_gather_body)
    out = run(idx, emb)
    return out.reshape(b, s, N_EMB)
